# Initial kernel scaffold; baseline (speedup 1.0000x reference)
import jax
import jax.numpy as jnp
from jax.experimental import pallas as pl


def kernel(x, edge_index, W1, b1, W2, b2, Wlin, blin):
    raise NotImplementedError("write your pallas kernel here")


# speedup vs baseline: 8.3187x; 8.3187x over previous
import functools

import jax
import jax.numpy as jnp
from jax import lax
from jax.experimental import pallas as pl
from jax.experimental.pallas import tpu as pltpu
from jax.experimental.pallas import tpu_sc as plsc

N = 10000
E = 320000
D = 128
NW = 32
EPW = 10240
C = 128
NCHUNK = EPW // C
NPAD = 10240
RPT = NPAD // 16

_MESH = plsc.VectorSubcoreMesh(core_axis_name="c", subcore_axis_name="s")


@functools.partial(
    pl.kernel,
    out_type=jax.ShapeDtypeStruct((2 * NPAD,), jnp.float32),
    scratch_types=[
        pltpu.VMEM((C,), jnp.int32),
        pltpu.VMEM((C,), jnp.float32),
        pltpu.VMEM_SHARED((NPAD,), jnp.float32),
    ],
    mesh=_MESH,
)
def _deg_kernel(dst_hbm, out_hbm, di, vals, acc):
    cid = lax.axis_index("c")
    sid = lax.axis_index("s")
    wid = cid * 16 + sid

    for j in range(C // 16):
        vals[pl.ds(j * 16, 16)] = jnp.zeros((16,), jnp.float32)
    for k in range(RPT // C):
        pltpu.sync_copy(vals, acc.at[pl.ds(sid * RPT + k * C, C)])
    for j in range(C // 16):
        vals[pl.ds(j * 16, 16)] = jnp.ones((16,), jnp.float32)
    plsc.subcore_barrier()

    base = wid * EPW

    def body(i, carry):
        pltpu.sync_copy(dst_hbm.at[pl.ds(base + i * C, C)], di)
        pltpu.sync_copy(vals, acc.at[di], add=True)
        return carry

    lax.fori_loop(0, NCHUNK, body, 0)
    plsc.subcore_barrier()
    pltpu.sync_copy(acc.at[pl.ds(sid * RPT, RPT)],
                    out_hbm.at[pl.ds(cid * NPAD + sid * RPT, RPT)])


@functools.partial(
    pl.kernel,
    out_type=jax.ShapeDtypeStruct((2 * NPAD, D), jnp.float32),
    scratch_types=[
        pltpu.VMEM((C,), jnp.int32),
        pltpu.VMEM((C,), jnp.int32),
        pltpu.VMEM((C, D), jnp.float32),
        pltpu.VMEM_SHARED((NPAD, D), jnp.float32),
        pltpu.SemaphoreType.DMA,
    ],
    mesh=_MESH,
)
def _agg_kernel(g_hbm, src_hbm, dst_hbm, out_hbm, si, di, rows, acc, sem):
    cid = lax.axis_index("c")
    sid = lax.axis_index("s")
    wid = cid * 16 + sid

    def zrow(i, carry):
        for j in range(D // 16):
            rows[i, pl.ds(j * 16, 16)] = jnp.zeros((16,), jnp.float32)
        return carry

    lax.fori_loop(0, C, zrow, 0)
    for k in range(RPT // C):
        pltpu.sync_copy(rows, acc.at[pl.ds(sid * RPT + k * C, C)])
    plsc.subcore_barrier()

    base = wid * EPW

    def body(i, carry):
        off = base + i * C
        pltpu.sync_copy(src_hbm.at[pl.ds(off, C)], si)
        pltpu.sync_copy(dst_hbm.at[pl.ds(off, C)], di)
        pltpu.async_copy(g_hbm.at[si], rows, sem).wait()
        pltpu.sync_copy(rows, acc.at[di], add=True)
        return carry

    lax.fori_loop(0, NCHUNK, body, 0)
    plsc.subcore_barrier()
    pltpu.sync_copy(acc.at[pl.ds(sid * RPT, RPT)],
                    out_hbm.at[pl.ds(cid * NPAD + sid * RPT, RPT)])


def _mm(a, w):
    return lax.dot_general(a, w, (((1,), (1,)), ((), ())),
                           preferred_element_type=jnp.float32,
                           precision=lax.Precision.HIGHEST)


def _tc1_body(degp_ref, x_ref, w1_ref, g1_ref, dinv_ref):
    deg = 1.0 + degp_ref[0] + degp_ref[1]
    dinv = lax.rsqrt(deg)
    dinv_ref[...] = dinv
    g1_ref[...] = dinv * _mm(x_ref[...], w1_ref[...])


def _tc2_body(sp_ref, g1_ref, dinv_ref, b1_ref, w2_ref, g2_ref):
    dinv = dinv_ref[...]
    s = sp_ref[0] + sp_ref[1]
    h = jnp.maximum(dinv * (s + g1_ref[...]) + b1_ref[...], 0.0)
    g2_ref[...] = dinv * _mm(h, w2_ref[...])


def _tc3_body(sp_ref, g2_ref, dinv_ref, b2_ref, wlin_ref, blin_ref, y_ref):
    dinv = dinv_ref[...]
    s = sp_ref[0] + sp_ref[1]
    h = jnp.maximum(dinv * (s + g2_ref[...]) + b2_ref[...], 0.0)
    y_ref[...] = jax.nn.sigmoid(_mm(h, wlin_ref[...]) + blin_ref[...])


_tc1 = pl.pallas_call(
    _tc1_body,
    out_shape=(jax.ShapeDtypeStruct((N, D), jnp.float32),
               jax.ShapeDtypeStruct((N, 1), jnp.float32)),
)
_tc2 = pl.pallas_call(
    _tc2_body,
    out_shape=jax.ShapeDtypeStruct((N, D), jnp.float32),
)
_tc3 = pl.pallas_call(
    _tc3_body,
    out_shape=jax.ShapeDtypeStruct((N, 64), jnp.float32),
)


def kernel(x, edge_index, W1, b1, W2, b2, Wlin, blin):
    src = edge_index[0].astype(jnp.int32)
    dst = edge_index[1].astype(jnp.int32)

    pad = EPW - E // NW
    src_p = jnp.concatenate(
        [src.reshape(NW, E // NW), jnp.zeros((NW, pad), jnp.int32)], axis=1
    ).reshape(-1)
    dst_p = jnp.concatenate(
        [dst.reshape(NW, E // NW), jnp.full((NW, pad), N, jnp.int32)], axis=1
    ).reshape(-1)

    degp = _deg_kernel(dst_p).reshape(2, NPAD, 1)[:, :N, :]
    g1, dinv = _tc1(degp, x, W1)
    s1 = _agg_kernel(g1, src_p, dst_p).reshape(2, NPAD, D)[:, :N, :]
    g2 = _tc2(s1, g1, dinv, b1.reshape(1, D), W2)
    s2 = _agg_kernel(g2, src_p, dst_p).reshape(2, NPAD, D)[:, :N, :]
    y = _tc3(s2, g2, dinv, b2.reshape(1, D), Wlin, blin.reshape(1, 64))
    return y
